# baseline (speedup 1.0000x reference)
import functools

import jax
import jax.numpy as jnp
from jax import lax
from jax.experimental import pallas as pl
from jax.experimental.pallas import tpu as pltpu
from jax.experimental.pallas import tpu_sc as plsc

_D = 128
_M = 500000
_B = 32
_K = 10000
_T = 0.07

_TM = 8192
_KC = 79
_KP = _KC * 128


def _prep_body(p_ref, kp_ref, ps_ref, kpn_ref, lpos_ref):
    p = p_ref[...]
    kp = kp_ref[...]
    pn = p / (jnp.sqrt(jnp.sum(p * p, axis=1, keepdims=True)) + 1e-12)
    kpn = kp / (jnp.sqrt(jnp.sum(kp * kp, axis=1, keepdims=True)) + 1e-12)
    ps_ref[...] = pn / _T
    kpn_ref[...] = kpn
    lpos_ref[...] = jnp.sum(pn * kpn, axis=1, keepdims=True) / _T


def _stream_body(ps_ref, kpn_ref, lab_ref, mem_ref, newm_ref, s_ref):
    i = pl.program_id(0)
    memb = mem_ref[...]
    ps = ps_ref[...]
    kpn = kpn_ref[...]
    labels = lab_ref[...]
    cols = lax.broadcasted_iota(jnp.int32, (1, _TM), 1) + i * _TM
    eq = labels == cols
    bidx = lax.broadcasted_iota(jnp.int32, (_B, _TM), 0)
    winner = jnp.max(jnp.where(eq, bidx, -1), axis=0, keepdims=True)
    onehot = jnp.where(eq & (bidx == winner), 1.0, 0.0)
    repl = lax.dot_general(kpn, onehot, (((0,), (0,)), ((), ())),
                           preferred_element_type=jnp.float32)
    newm_ref[...] = jnp.where(winner >= 0, repl, memb)
    s_ref[...] = lax.dot_general(ps, memb, (((1,), (0,)), ((), ())),
                                 preferred_element_type=jnp.float32)


def _loss_body(logits_ref, loss_ref):
    lg = logits_ref[...]
    m = jnp.max(lg, axis=1, keepdims=True)
    lse = jnp.log(jnp.sum(jnp.exp(lg - m), axis=1, keepdims=True)) + m
    lv = lse - lg[:, 0:1]
    loss_ref[...] = jnp.sum(lv, axis=0, keepdims=True) / _B


def _make_gather():
    info = plsc.get_sparse_core_info()
    nc = info.num_cores
    mesh = plsc.VectorSubcoreMesh(core_axis_name="c", subcore_axis_name="s")

    @functools.partial(
        pl.kernel,
        out_type=jax.ShapeDtypeStruct((_B, _KC, 128), jnp.float32),
        mesh=mesh,
        scratch_types=[
            pltpu.VMEM((_KC, 128), jnp.int32),
            pltpu.VMEM((_KC, 128), jnp.float32),
            pltpu.SemaphoreType.DMA,
        ],
    )
    def gather_k(sflat_hbm, fidx_hbm, out_hbm, idx_v, rows_v, sem):
        wid = lax.axis_index("s") * nc + lax.axis_index("c")
        pltpu.sync_copy(fidx_hbm.at[wid], idx_v)

        def fire(j, carry):
            pltpu.async_copy(sflat_hbm.at[idx_v.at[j]], rows_v.at[j], sem)
            return carry

        lax.fori_loop(0, _KC, fire, 0)

        def drain(j, carry):
            pltpu.make_async_copy(
                sflat_hbm.at[idx_v.at[j]], rows_v.at[j], sem).wait()
            return carry

        lax.fori_loop(0, _KC, drain, 0)
        pltpu.sync_copy(rows_v, out_hbm.at[wid])

    return gather_k


def kernel(projectors, key_projectors, mem, neg_idx, key_labels):
    f32 = jnp.float32
    ps, kpn, lpos_t = pl.pallas_call(
        _prep_body,
        out_shape=[
            jax.ShapeDtypeStruct((_B, _D), f32),
            jax.ShapeDtypeStruct((_B, _D), f32),
            jax.ShapeDtypeStruct((_B, 1), f32),
        ],
    )(projectors, key_projectors)

    labels2 = key_labels.astype(jnp.int32).reshape(_B, 1)
    n_tiles = (_M + _TM - 1) // _TM
    newm, s = pl.pallas_call(
        _stream_body,
        grid=(n_tiles,),
        in_specs=[
            pl.BlockSpec((_B, _D), lambda i: (0, 0)),
            pl.BlockSpec((_B, _D), lambda i: (0, 0)),
            pl.BlockSpec((_B, 1), lambda i: (0, 0)),
            pl.BlockSpec((_D, _TM), lambda i: (0, i)),
        ],
        out_specs=[
            pl.BlockSpec((_D, _TM), lambda i: (0, i)),
            pl.BlockSpec((_B, _TM), lambda i: (0, i)),
        ],
        out_shape=[
            jax.ShapeDtypeStruct((_D, _M), f32),
            jax.ShapeDtypeStruct((_B, _M), f32),
        ],
        compiler_params=pltpu.CompilerParams(
            dimension_semantics=("parallel",)),
    )(ps, kpn, labels2, mem)

    sflat = s.reshape(_B * _M)
    fidx = neg_idx.astype(jnp.int32) + (jnp.arange(_B, dtype=jnp.int32) * _M)[:, None]
    fidx = jnp.pad(fidx, ((0, 0), (0, _KP - _K))).reshape(_B, _KC, 128)
    lneg_p = _make_gather()(sflat, fidx)
    lneg = lneg_p.reshape(_B, _KP)[:, :_K]

    logits = jnp.concatenate([lpos_t, lneg], axis=1)
    loss = pl.pallas_call(
        _loss_body,
        out_shape=jax.ShapeDtypeStruct((1, 1), f32),
    )(logits)
    return loss.reshape(()), logits, newm


# speedup vs baseline: 4.9777x; 4.9777x over previous
import functools

import jax
import jax.numpy as jnp
from jax import lax
from jax.experimental import pallas as pl
from jax.experimental.pallas import tpu as pltpu
from jax.experimental.pallas import tpu_sc as plsc

_D = 128
_M = 500000
_B = 32
_K = 10000
_T = 0.07

_TR = 8192
_NT = 62
_MP = _NT * _TR
_QP = _MP // 128
_KC = 79
_KP = _KC * 128


def _prep_body(p_ref, kp_ref, ps_ref, kpn_ref, lpos_ref):
    p = p_ref[...]
    kp = kp_ref[...]
    pn = p / (jnp.sqrt(jnp.sum(p * p, axis=1, keepdims=True)) + 1e-12)
    kpn = kp / (jnp.sqrt(jnp.sum(kp * kp, axis=1, keepdims=True)) + 1e-12)
    ps_ref[...] = pn / _T
    kpn_ref[...] = kpn
    lpos_ref[...] = jnp.sum(pn * kpn, axis=1, keepdims=True) / _T


def _stream_body(ps_ref, kpn_ref, lab_ref, memt_ref, newmt_ref, s_ref):
    i = pl.program_id(0)
    memb = memt_ref[...]
    ps = ps_ref[...]
    kpn = kpn_ref[...]
    labt = lab_ref[0:1, :]
    rowid = lax.broadcasted_iota(jnp.int32, (_TR, 1), 0) + i * _TR
    eq = rowid == labt
    bidx = lax.broadcasted_iota(jnp.int32, (_TR, _B), 1)
    winner = jnp.max(jnp.where(eq, bidx, -1), axis=1, keepdims=True)
    onehot = jnp.where(eq & (bidx == winner), 1.0, 0.0)
    repl = lax.dot_general(onehot, kpn, (((1,), (0,)), ((), ())),
                           preferred_element_type=jnp.float32)
    newmt_ref[...] = jnp.where(winner >= 0, repl, memb)
    st = lax.dot_general(ps, memb, (((1,), (1,)), ((), ())),
                         preferred_element_type=jnp.float32)
    s_ref[...] = st.reshape(_B, _TR // 128, 128)


def _loss_body(logits_ref, loss_ref):
    lg = logits_ref[...]
    m = jnp.max(lg, axis=1, keepdims=True)
    lse = jnp.log(jnp.sum(jnp.exp(lg - m), axis=1, keepdims=True)) + m
    lv = lse - lg[:, 0:1]
    loss_ref[...] = jnp.sum(lv, axis=0, keepdims=True) / _B


def _make_gather():
    info = plsc.get_sparse_core_info()
    nc = info.num_cores
    mesh = plsc.VectorSubcoreMesh(core_axis_name="c", subcore_axis_name="s")

    @functools.partial(
        pl.kernel,
        out_type=jax.ShapeDtypeStruct((_B, _KC, 128), jnp.float32),
        mesh=mesh,
        scratch_types=[
            pltpu.VMEM((_KC, 128), jnp.int32),
            pltpu.VMEM((_KC, 128), jnp.float32),
            pltpu.SemaphoreType.DMA,
        ],
    )
    def gather_k(sflat_hbm, fidx_hbm, out_hbm, idx_v, rows_v, sem):
        wid = lax.axis_index("s") * nc + lax.axis_index("c")
        pltpu.sync_copy(fidx_hbm.at[wid], idx_v)

        def fire(j, carry):
            pltpu.async_copy(sflat_hbm.at[idx_v.at[j]], rows_v.at[j], sem)
            return carry

        lax.fori_loop(0, _KC, fire, 0)

        def drain(j, carry):
            pltpu.make_async_copy(
                sflat_hbm.at[idx_v.at[j]], rows_v.at[j], sem).wait()
            return carry

        lax.fori_loop(0, _KC, drain, 0)
        pltpu.sync_copy(rows_v, out_hbm.at[wid])

    return gather_k


def kernel(projectors, key_projectors, mem, neg_idx, key_labels):
    f32 = jnp.float32
    ps, kpn, lpos_t = pl.pallas_call(
        _prep_body,
        out_shape=[
            jax.ShapeDtypeStruct((_B, _D), f32),
            jax.ShapeDtypeStruct((_B, _D), f32),
            jax.ShapeDtypeStruct((_B, 1), f32),
        ],
    )(projectors, key_projectors)

    memt = mem.T
    labels2 = jnp.broadcast_to(
        key_labels.astype(jnp.int32).reshape(1, _B), (8, _B))
    newmt, s6 = pl.pallas_call(
        _stream_body,
        grid=(_NT,),
        in_specs=[
            pl.BlockSpec((_B, _D), lambda i: (0, 0)),
            pl.BlockSpec((_B, _D), lambda i: (0, 0)),
            pl.BlockSpec((8, _B), lambda i: (0, 0)),
            pl.BlockSpec((_TR, _D), lambda i: (i, 0)),
        ],
        out_specs=[
            pl.BlockSpec((_TR, _D), lambda i: (i, 0)),
            pl.BlockSpec((_B, _TR // 128, 128), lambda i: (0, i, 0)),
        ],
        out_shape=[
            jax.ShapeDtypeStruct((_M, _D), f32),
            jax.ShapeDtypeStruct((_B, _QP, 128), f32),
        ],
        compiler_params=pltpu.CompilerParams(
            dimension_semantics=("parallel",)),
    )(ps, kpn, labels2, memt)

    sflat = s6.reshape(_B * _MP)
    fidx = neg_idx.astype(jnp.int32) + (jnp.arange(_B, dtype=jnp.int32) * _MP)[:, None]
    fidx = jnp.pad(fidx, ((0, 0), (0, _KP - _K))).reshape(_B, _KC, 128)
    lneg_p = _make_gather()(sflat, fidx)
    lneg = lneg_p.reshape(_B, _KP)[:, :_K]

    logits = jnp.concatenate([lpos_t, lneg], axis=1)
    loss = pl.pallas_call(
        _loss_body,
        out_shape=jax.ShapeDtypeStruct((1, 1), f32),
    )(logits)
    return loss.reshape(()), logits, newmt.T
